# baseline (speedup 1.0000x reference)
import functools

import jax
import jax.numpy as jnp
from jax import lax
from jax.experimental import pallas as pl
from jax.experimental.pallas import tpu as pltpu
from jax.experimental.pallas import tpu_sc as plsc

E = 8
K = 2
BM_R = 512
BM = 256
RC = 64
CT = 16

def _take16(vec, idx):
    dnums = lax.GatherDimensionNumbers(
        offset_dims=(), collapsed_slice_dims=(0,), start_index_map=(0,))
    return lax.gather(vec, idx[:, None], dnums, (1,),
                      mode=lax.GatherScatterMode.PROMISE_IN_BOUNDS)


_info = plsc.get_sparse_core_info()
_NC, _NS, _L = _info.num_cores, _info.num_subcores, _info.num_lanes
_NW = _NC * _NS


def _router_body(br_ref, x_ref, wr_ref, logits_ref, pi_ref, ii_ref):
    x = x_ref[...]
    wr = wr_ref[...]
    logits = jax.lax.dot_general(
        x, wr, (((1,), (1,)), ((), ())),
        preferred_element_type=jnp.float32) + br_ref[...]
    logits_ref[...] = logits
    cols = jax.lax.broadcasted_iota(jnp.int32, logits.shape, 1)
    big = jnp.int32(2 ** 30)
    m1 = jnp.max(logits, axis=-1, keepdims=True)
    i1 = jnp.min(jnp.where(logits == m1, cols, big), axis=-1, keepdims=True)
    l2 = jnp.where(cols == i1, -jnp.inf, logits)
    m2 = jnp.max(l2, axis=-1, keepdims=True)
    i2 = jnp.min(jnp.where(l2 == m2, cols, big), axis=-1, keepdims=True)
    p1 = 1.0 / (1.0 + jnp.exp(m2 - m1))
    p2 = 1.0 - p1
    pi_ref[...] = jnp.where(cols == 0, p1, jnp.where(cols == 1, p2, 0.0))
    ii_ref[...] = jnp.where(cols == 0, i1, jnp.where(cols == 1, i2, 0))


def _router(x_flat, Wr, br):
    T, D = x_flat.shape
    return pl.pallas_call(
        _router_body,
        grid=(T // BM_R,),
        in_specs=[
            pl.BlockSpec((1, E), lambda g: (0, 0)),
            pl.BlockSpec((BM_R, D), lambda g: (g, 0)),
            pl.BlockSpec((E, D), lambda g: (0, 0)),
        ],
        out_specs=[
            pl.BlockSpec((BM_R, E), lambda g: (g, 0)),
            pl.BlockSpec((BM_R, E), lambda g: (g, 0)),
            pl.BlockSpec((BM_R, E), lambda g: (g, 0)),
        ],
        out_shape=[
            jax.ShapeDtypeStruct((T, E), jnp.float32),
            jax.ShapeDtypeStruct((T, E), jnp.float32),
            jax.ShapeDtypeStruct((T, E), jnp.int32),
        ],
    )(br.reshape(1, E), x_flat, Wr)


def _rank_body(ii_ref, rank_ref, counts_ref, carry):
    g = pl.program_id(0)

    @pl.when(g == 0)
    def _():
        carry[...] = jnp.zeros_like(carry)

    nb = pl.num_programs(0) // K
    col = g // nb
    e_blk = ii_ref[...]
    cols8 = jax.lax.broadcasted_iota(jnp.int32, e_blk.shape, 1)
    e_vec = jnp.sum(jnp.where(cols8 == col, e_blk, 0), axis=1, keepdims=True)
    onehot = (cols8 == e_vec).astype(jnp.float32)
    r_i = jax.lax.broadcasted_iota(jnp.int32, (BM_R, BM_R), 0)
    c_i = jax.lax.broadcasted_iota(jnp.int32, (BM_R, BM_R), 1)
    tri = (c_i < r_i).astype(jnp.float32)
    rank_blk = jax.lax.dot(tri, onehot,
                           preferred_element_type=jnp.float32) + carry[...]
    rank_vec = jnp.sum(rank_blk * onehot, axis=1, keepdims=True)
    rank_ref[...] = rank_vec.astype(jnp.int32)
    carry[...] = carry[...] + jnp.sum(onehot, axis=0, keepdims=True)
    counts_ref[...] = carry[...].astype(jnp.int32)


def _rank(ii):
    T = ii.shape[0]
    TK = T * K
    nb = T // BM_R
    return pl.pallas_call(
        _rank_body,
        grid=(nb * K,),
        in_specs=[pl.BlockSpec((BM_R, E), lambda g: (lax.rem(g, nb), 0))],
        out_specs=[
            pl.BlockSpec((BM_R, 1), lambda g: (g, 0)),
            pl.BlockSpec((1, E), lambda g: (0, 0)),
        ],
        out_shape=[
            jax.ShapeDtypeStruct((TK, 1), jnp.int32),
            jax.ShapeDtypeStruct((1, E), jnp.int32),
        ],
        scratch_shapes=[pltpu.VMEM((1, E), jnp.float32)],
    )(ii)


def _dispatch(x_i32, e_sm, rank_sm, pstart16, TKPAD):
    T, D = x_i32.shape
    TK = e_sm.shape[0]
    per_w = TK // _NW
    nch = per_w // RC

    @functools.partial(
        pl.kernel,
        out_type=jax.ShapeDtypeStruct((TKPAD, D), jnp.int32),
        mesh=plsc.VectorSubcoreMesh(core_axis_name="c", subcore_axis_name="s"),
        scratch_types=[
            pltpu.VMEM((RC, D), jnp.int32),
            pltpu.VMEM((RC,), jnp.int32),
            pltpu.VMEM((RC,), jnp.int32),
            pltpu.VMEM((RC,), jnp.int32),
            pltpu.VMEM((16,), jnp.int32),
            pltpu.SemaphoreType.DMA,
        ],
    )
    def k(x_hbm, e_hbm, r_hbm, ps_hbm, xs_hbm,
          rows_v, dst_v, e_v, r_v, ps_v, sem):
        wid = lax.axis_index("s") * _NC + lax.axis_index("c")
        pltpu.sync_copy(ps_hbm, ps_v)
        ps = ps_v[...]
        for c in range(nch):
            base = wid * per_w + c * RC
            src = jnp.where(base < T, base, base - T)
            pltpu.sync_copy(e_hbm.at[pl.ds(base, RC)], e_v)
            pltpu.sync_copy(r_hbm.at[pl.ds(base, RC)], r_v)
            pltpu.sync_copy(x_hbm.at[pl.ds(src, RC)], rows_v)
            for j in range(RC // _L):
                sl = pl.ds(j * _L, _L)
                dst_v[sl] = r_v[sl] + _take16(ps, e_v[sl])
            pltpu.async_copy(rows_v, xs_hbm.at[dst_v], sem).wait()

    return k(x_i32, e_sm, rank_sm, pstart16)


def _combine(ys, p_sm, e_sm, rank_sm, pstart16, T):
    TKPAD, D = ys.shape
    per_w = T // _NW
    nch = per_w // CT

    @functools.partial(
        pl.kernel,
        out_type=jax.ShapeDtypeStruct((T, D), jnp.float32),
        mesh=plsc.VectorSubcoreMesh(core_axis_name="c", subcore_axis_name="s"),
        scratch_types=[
            pltpu.VMEM((CT, D), jnp.float32),
            pltpu.VMEM((CT, D), jnp.float32),
            pltpu.VMEM((CT,), jnp.int32),
            pltpu.VMEM((CT,), jnp.int32),
            pltpu.VMEM((CT,), jnp.int32),
            pltpu.VMEM((CT,), jnp.int32),
            pltpu.VMEM((CT,), jnp.float32),
            pltpu.VMEM((CT,), jnp.float32),
            pltpu.VMEM((16,), jnp.int32),
            pltpu.SemaphoreType.DMA,
            pltpu.SemaphoreType.DMA,
        ],
    )
    def k(ys_hbm, p_hbm, e_hbm, r_hbm, ps_hbm, out_hbm,
          a_v, b_v, d0_v, d1_v, e_v, r_v, p0_v, p1_v, ps_v, sem0, sem1):
        wid = lax.axis_index("s") * _NC + lax.axis_index("c")
        pltpu.sync_copy(ps_hbm, ps_v)
        ps = ps_v[...]
        for c in range(nch):
            tbase = wid * per_w + c * CT
            pltpu.sync_copy(e_hbm.at[pl.ds(tbase, CT)], e_v)
            pltpu.sync_copy(r_hbm.at[pl.ds(tbase, CT)], r_v)
            pltpu.sync_copy(p_hbm.at[pl.ds(tbase, CT)], p0_v)
            d0_v[...] = r_v[...] + _take16(ps, e_v[...])
            pltpu.sync_copy(e_hbm.at[pl.ds(T + tbase, CT)], e_v)
            pltpu.sync_copy(r_hbm.at[pl.ds(T + tbase, CT)], r_v)
            pltpu.sync_copy(p_hbm.at[pl.ds(T + tbase, CT)], p1_v)
            d1_v[...] = r_v[...] + _take16(ps, e_v[...])
            cp0 = pltpu.async_copy(ys_hbm.at[d0_v], a_v, sem0)
            cp1 = pltpu.async_copy(ys_hbm.at[d1_v], b_v, sem1)
            cp0.wait()
            cp1.wait()
            p0 = p0_v[...]
            p1 = p1_v[...]
            for row in range(CT):
                ridx = jnp.full((_L,), row, jnp.int32)
                pa = _take16(p0, ridx)
                pb = _take16(p1, ridx)

                def body(i, _, row=row, pa=pa, pb=pb):
                    for u in range(4):
                        sl = pl.ds(i * (4 * _L) + u * _L, _L)
                        a_v[row, sl] = pa * a_v[row, sl] + pb * b_v[row, sl]
                    return 0
                lax.fori_loop(0, D // (4 * _L), body, 0)
            pltpu.sync_copy(a_v, out_hbm.at[pl.ds(tbase, CT)])

    return k(ys, p_sm, e_sm, rank_sm, pstart16)


def _gemm_body(eot_ref, nv_ref, xs_ref, w1_ref, w2_ref, ys_ref):
    g = pl.program_id(0)

    @pl.when(g < nv_ref[0])
    def _():
        xt = xs_ref[...]
        h = jax.lax.dot_general(
            xt, w1_ref[0], (((1,), (1,)), ((), ())),
            preferred_element_type=jnp.float32)
        h = 0.5 * h * (1.0 + jax.lax.erf(h * 0.7071067811865476))
        y = jax.lax.dot_general(
            h.astype(jnp.bfloat16), w2_ref[0], (((1,), (1,)), ((), ())),
            preferred_element_type=jnp.float32)
        ys_ref[...] = y


def _grouped_gemm(eot, nv, xs, W1, W2):
    TKPAD, D = xs.shape
    H = W1.shape[1]
    NT = TKPAD // BM
    grid_spec = pltpu.PrefetchScalarGridSpec(
        num_scalar_prefetch=2,
        grid=(NT,),
        in_specs=[
            pl.BlockSpec((BM, D), lambda g, eot, nv: (g, 0)),
            pl.BlockSpec((1, H, D), lambda g, eot, nv: (eot[g], 0, 0)),
            pl.BlockSpec((1, D, H), lambda g, eot, nv: (eot[g], 0, 0)),
        ],
        out_specs=pl.BlockSpec((BM, D), lambda g, eot, nv: (g, 0)),
    )
    return pl.pallas_call(
        _gemm_body,
        grid_spec=grid_spec,
        out_shape=jax.ShapeDtypeStruct((TKPAD, D), jnp.float32),
    )(eot, nv, xs, W1, W2)


def kernel(x, Wr, br, W1, W2):
    b, s, d = x.shape
    x_flat = x.reshape(-1, d)
    T = x_flat.shape[0]
    TK = T * K
    TKPAD = TK + E * BM

    router_logits, pi, ii = _router(x_flat, Wr, br)
    rank, counts2 = _rank(ii)
    counts = counts2.reshape(E)
    rank_sm = rank.reshape(TK)
    e_sm = jnp.concatenate([ii[:, 0], ii[:, 1]])
    p_sm = jnp.concatenate([pi[:, 0], pi[:, 1]])

    padded_counts = ((counts + BM - 1) // BM) * BM
    pstart = jnp.concatenate(
        [jnp.zeros((1,), jnp.int32), jnp.cumsum(padded_counts)[:-1]])
    pend = pstart + padded_counts
    tile_starts = jnp.arange(TKPAD // BM, dtype=jnp.int32) * BM
    eot = jnp.minimum(
        jnp.searchsorted(pend, tile_starts, side='right').astype(jnp.int32),
        E - 1)
    nv = (pend[E - 1] // BM).reshape(1)
    pstart16 = jnp.concatenate([pstart, jnp.zeros((8,), jnp.int32)])

    x_i32 = lax.bitcast_convert_type(
        x_flat.astype(jnp.bfloat16).reshape(T, d // 2, 2), jnp.int32)
    xs_i32 = _dispatch(x_i32, e_sm, rank_sm, pstart16, TKPAD)
    xs = lax.bitcast_convert_type(
        xs_i32, jnp.bfloat16).reshape(TKPAD, d)
    ys = _grouped_gemm(eot, nv, xs, W1.astype(jnp.bfloat16),
                       W2.astype(jnp.bfloat16))
    out_flat = _combine(ys, p_sm, e_sm, rank_sm, pstart16, T)
    return out_flat.reshape(b, s, d), router_logits
